# baseline (speedup 1.0000x reference)
import jax
import jax.numpy as jnp
from jax import lax
from jax.experimental import pallas as pl
from jax.experimental.pallas import tpu as pltpu
from jax.experimental.pallas import tpu_sc as plsc

RES = 512
DIM = 16
ODIM = 3 * DIM
QW = 32
N_PTS = 2097152
NC, NS, LANES = 2, 16, 16
NW = NC * NS
PTS_W = N_PTS // NW
B = 128
STEPS = PTS_W // B
HALF = STEPS // 2
GROUPS = B // LANES


def _pix(c):
    p = jnp.clip(c * 128.0 + 255.5, 0.0, float(RES - 1))
    i0 = p.astype(jnp.int32)
    f = p - i0.astype(jnp.float32)
    return i0, f


def _tec_body(loc, table, out, cbuf, idxv, wv, gath, ob0, ob1,
              semc0, semc1, semg0, semg1, semo0, semo1):
    wid = lax.axis_index("s") * NC + lax.axis_index("c")
    semc = (semc0, semc1)
    semg = (semg0, semg1)
    semo = (semo0, semo1)
    obs = (ob0, ob1)

    def fire_coords(q, s):
        base = wid * PTS_W + s * B
        pltpu.async_copy(loc.at[pl.ds(base, B), :], cbuf.at[q], semc[q])

    def wait_coords(q):
        pltpu.make_async_copy(loc.at[pl.ds(0, B), :], cbuf.at[q],
                              semc[q]).wait()

    def compute_idx(q):
        for g in range(GROUPS):
            sl = pl.ds(g * LANES, LANES)
            pid = lax.iota(jnp.int32, LANES) + g * LANES
            qv = jnp.full((LANES,), q, jnp.int32)
            x = plsc.load_gather(cbuf, (qv, pid, jnp.full((LANES,), 0,
                                                          jnp.int32)))
            y = plsc.load_gather(cbuf, (qv, pid, jnp.full((LANES,), 1,
                                                          jnp.int32)))
            z = plsc.load_gather(cbuf, (qv, pid, jnp.full((LANES,), 2,
                                                          jnp.int32)))
            xi0, xf = _pix(x)
            yi0, yf = _pix(y)
            zi0, zf = _pix(z)
            specs = ((xi0, xf, yi0, yf),
                     (yi0, yf, zi0, zf),
                     (xi0, xf, zi0, zf))
            for p, (ci0, cf, ri0, rf) in enumerate(specs):
                idxv[q * 3 + p, sl] = (p * RES * RES) + ri0 * RES + ci0
                cw0 = 1.0 - cf
                rw0 = 1.0 - rf
                o = q * 12 + 4 * p
                wv[o + 0, sl] = rw0 * cw0
                wv[o + 1, sl] = rw0 * cf
                wv[o + 2, sl] = rf * cw0
                wv[o + 3, sl] = rf * cf

    def fire_gathers(q):
        for p in range(3):
            r = q * 3 + p
            pltpu.async_copy(table.at[idxv.at[r]],
                             gath.at[pl.ds(r * B, B)], semg[q])

    def wait_gathers(q):
        for p in range(3):
            r = q * 3 + p
            pltpu.make_async_copy(table.at[idxv.at[r]],
                                  gath.at[pl.ds(r * B, B)], semg[q]).wait()

    def wsum(q):
        ob = obs[q]

        def wsum_fn(g, _):
            sl = pl.ds(g * LANES, LANES)
            pid = lax.iota(jnp.int32, LANES) + g * LANES
            for pi in range(3):
                w = [wv[q * 12 + 4 * pi + t, sl] for t in range(4)]
                row = (q * 3 + pi) * B + pid
                for cp in range(DIM // 2):
                    av = []
                    bv = []
                    for t in range(4):
                        cv = jnp.full((LANES,), t * (DIM // 2) + cp, jnp.int32)
                        v = plsc.load_gather(gath, (row, cv))
                        a, b = plsc.unpack(
                            plsc.bitcast(v, jnp.bfloat16),
                            format=plsc.PackFormat.INTERLEAVED)
                        av.append(a)
                        bv.append(b)
                    acca = av[0] * w[0] + av[1] * w[1] + av[2] * w[2] \
                        + av[3] * w[3]
                    accb = bv[0] * w[0] + bv[1] * w[1] + bv[2] * w[2] \
                        + bv[3] * w[3]
                    ov = pid * ODIM + (pi * DIM + 2 * cp)
                    plsc.store_scatter(ob, (ov,), acca)
                    plsc.store_scatter(ob, (ov + 1,), accb)
            return 0

        lax.fori_loop(0, GROUPS, wsum_fn, 0)

    def fire_out(q, s):
        base = (wid * PTS_W + s * B) * ODIM
        pltpu.async_copy(obs[q], out.at[pl.ds(base, B * ODIM)], semo[q])

    def wait_out(q):
        pltpu.make_async_copy(obs[q], out.at[pl.ds(0, B * ODIM)],
                              semo[q]).wait()

    fire_coords(0, 0)
    wait_coords(0)
    compute_idx(0)
    fire_gathers(0)
    fire_coords(1, 1)
    wait_coords(1)
    compute_idx(1)
    fire_gathers(1)
    fire_coords(0, 2)
    wait_gathers(0)
    wsum(0)
    fire_out(0, 0)

    def pair_body(i2, _):
        a = 2 * i2 + 1
        wait_coords(0)
        compute_idx(0)
        fire_gathers(0)
        fire_coords(1, a + 2)
        wait_gathers(1)

        @pl.when(i2 >= 1)
        def _():
            wait_out(1)

        wsum(1)
        fire_out(1, a)

        b = a + 1
        wait_coords(1)
        compute_idx(1)
        fire_gathers(1)

        @pl.when(i2 < HALF - 2)
        def _():
            fire_coords(0, b + 2)

        wait_gathers(0)
        wait_out(0)
        wsum(0)
        fire_out(0, b)
        return 0

    lax.fori_loop(0, HALF - 1, pair_body, 0)

    wait_gathers(1)
    wait_out(1)
    wsum(1)
    fire_out(1, STEPS - 1)
    wait_out(0)
    wait_out(1)


@jax.jit
def _sc_sample(loc, table):
    mesh = plsc.VectorSubcoreMesh(core_axis_name="c", subcore_axis_name="s",
                                  num_cores=NC, num_subcores=NS)
    f = pl.kernel(
        _tec_body,
        out_type=jax.ShapeDtypeStruct((N_PTS * ODIM,), jnp.float32),
        mesh=mesh,
        scratch_types=[
            pltpu.VMEM((2, B, 3), jnp.float32),
            pltpu.VMEM((2 * 3, B), jnp.int32),
            pltpu.VMEM((2 * 12, B), jnp.float32),
            pltpu.VMEM((2 * 3 * B, QW), jnp.int32),
            pltpu.VMEM((B * ODIM,), jnp.float32),
            pltpu.VMEM((B * ODIM,), jnp.float32),
            pltpu.SemaphoreType.DMA,
            pltpu.SemaphoreType.DMA,
            pltpu.SemaphoreType.DMA,
            pltpu.SemaphoreType.DMA,
            pltpu.SemaphoreType.DMA,
            pltpu.SemaphoreType.DMA,
        ],
        compiler_params=pltpu.CompilerParams(use_tc_tiling_on_sc=False,
                                             needs_layout_passes=False),
    )
    return f(loc, table)


def kernel(loc, triplane):
    t = triplane.astype(jnp.bfloat16)
    tx = jnp.concatenate([t[:, :, :, 1:], t[:, :, :, -1:]], axis=3)
    ty = jnp.concatenate([t[:, :, 1:], t[:, :, -1:]], axis=2)
    txy = jnp.concatenate([tx[:, :, 1:], tx[:, :, -1:]], axis=2)
    q = jnp.stack([t, tx, ty, txy], axis=1)
    q = q.reshape(3, 4 * DIM, RES * RES)
    eye = jnp.eye(4 * DIM, dtype=jnp.bfloat16)
    tq = jnp.einsum("pcn,cd->pnd", q, eye,
                    preferred_element_type=jnp.bfloat16)
    tq = jax.lax.bitcast_convert_type(
        tq.reshape(3 * RES * RES, QW, 2), jnp.int32)
    out = _sc_sample(loc, tq)
    return out.reshape(N_PTS, ODIM)


# speedup vs baseline: 1.0594x; 1.0594x over previous
import jax
import jax.numpy as jnp
from jax import lax
from jax.experimental import pallas as pl
from jax.experimental.pallas import tpu as pltpu
from jax.experimental.pallas import tpu_sc as plsc

RES = 512
DIM = 16
ODIM = 3 * DIM
QW = 32
N_PTS = 2097152
NC, NS, LANES = 2, 16, 16
NW = NC * NS
PTS_W = N_PTS // NW
B = 128
STEPS = PTS_W // B
HALF = STEPS // 2
GROUPS = B // LANES


def _pix(c):
    p = jnp.clip(c * 128.0 + 255.5, 0.0, float(RES - 1))
    i0 = p.astype(jnp.int32)
    f = p - i0.astype(jnp.float32)
    return i0, f


def _tec_body(loc, table, out, cbuf, idxv, wv, gath, ob0, ob1,
              semc0, semc1, semg0, semg1, semo0, semo1):
    wid = lax.axis_index("s") * NC + lax.axis_index("c")
    semc = (semc0, semc1)
    semg = (semg0, semg1)
    semo = (semo0, semo1)
    obs = (ob0, ob1)

    def fire_coords(q, s):
        base = wid * PTS_W + s * B
        pltpu.async_copy(loc.at[pl.ds(base, B), :], cbuf.at[q], semc[q])

    def wait_coords(q):
        pltpu.make_async_copy(loc.at[pl.ds(0, B), :], cbuf.at[q],
                              semc[q]).wait()

    def compute_idx(q):
        for g in range(GROUPS):
            sl = pl.ds(g * LANES, LANES)
            pid = lax.iota(jnp.int32, LANES) + g * LANES
            qv = jnp.full((LANES,), q, jnp.int32)
            x = plsc.load_gather(cbuf, (qv, pid, jnp.full((LANES,), 0,
                                                          jnp.int32)))
            y = plsc.load_gather(cbuf, (qv, pid, jnp.full((LANES,), 1,
                                                          jnp.int32)))
            z = plsc.load_gather(cbuf, (qv, pid, jnp.full((LANES,), 2,
                                                          jnp.int32)))
            xi0, xf = _pix(x)
            yi0, yf = _pix(y)
            zi0, zf = _pix(z)
            specs = ((xi0, xf, yi0, yf),
                     (yi0, yf, zi0, zf),
                     (xi0, xf, zi0, zf))
            for p, (ci0, cf, ri0, rf) in enumerate(specs):
                idxv[q * 3 + p, sl] = (p * RES * RES) + ri0 * RES + ci0
                cw0 = 1.0 - cf
                rw0 = 1.0 - rf
                o = q * 12 + 4 * p
                wv[o + 0, sl] = rw0 * cw0
                wv[o + 1, sl] = rw0 * cf
                wv[o + 2, sl] = rf * cw0
                wv[o + 3, sl] = rf * cf

    def fire_gathers(q):
        for p in range(3):
            r = q * 3 + p
            pltpu.async_copy(table.at[idxv.at[r]],
                             gath.at[pl.ds(r * B, B)], semg[q])

    def wait_gathers(q):
        for p in range(3):
            r = q * 3 + p
            pltpu.make_async_copy(table.at[idxv.at[r]],
                                  gath.at[pl.ds(r * B, B)], semg[q]).wait()

    def wsum(q):
        ob = obs[q]

        def wsum_fn(g, _):
            sl = pl.ds(g * LANES, LANES)
            pid = lax.iota(jnp.int32, LANES) + g * LANES
            for pi in range(3):
                w = [wv[q * 12 + 4 * pi + t, sl] for t in range(4)]
                row = (q * 3 + pi) * B + pid
                for cp in range(DIM // 2):
                    av = []
                    bv = []
                    for t in range(4):
                        cv = jnp.full((LANES,), t * (DIM // 2) + cp, jnp.int32)
                        v = plsc.load_gather(gath, (row, cv))
                        a, b = plsc.unpack(
                            plsc.bitcast(v, jnp.bfloat16),
                            format=plsc.PackFormat.INTERLEAVED)
                        av.append(a)
                        bv.append(b)
                    acca = av[0] * w[0] + av[1] * w[1] + av[2] * w[2] \
                        + av[3] * w[3]
                    accb = bv[0] * w[0] + bv[1] * w[1] + bv[2] * w[2] \
                        + bv[3] * w[3]
                    ov = pid * ODIM + (pi * DIM + 2 * cp)
                    plsc.store_scatter(ob, (ov,), acca)
                    plsc.store_scatter(ob, (ov + 1,), accb)
            return 0

        lax.fori_loop(0, GROUPS, wsum_fn, 0)

    def fire_out(q, s):
        base = (wid * PTS_W + s * B) * ODIM
        pltpu.async_copy(obs[q], out.at[pl.ds(base, B * ODIM)], semo[q])

    def wait_out(q):
        pltpu.make_async_copy(obs[q], out.at[pl.ds(0, B * ODIM)],
                              semo[q]).wait()

    fire_coords(0, 0)
    wait_coords(0)
    compute_idx(0)
    fire_gathers(0)
    fire_coords(1, 1)
    wait_coords(1)
    compute_idx(1)
    fire_gathers(1)
    fire_coords(0, 2)
    wait_gathers(0)
    wsum(0)
    fire_out(0, 0)

    def pair_body(i2, _):
        a = 2 * i2 + 1
        wait_coords(0)
        compute_idx(0)
        fire_gathers(0)
        fire_coords(1, a + 2)
        wait_gathers(1)

        @pl.when(i2 >= 1)
        def _():
            wait_out(1)

        wsum(1)
        fire_out(1, a)

        b = a + 1
        wait_coords(1)
        compute_idx(1)
        fire_gathers(1)

        @pl.when(i2 < HALF - 2)
        def _():
            fire_coords(0, b + 2)

        wait_gathers(0)
        wait_out(0)
        wsum(0)
        fire_out(0, b)
        return 0

    lax.fori_loop(0, HALF - 1, pair_body, 0)

    wait_gathers(1)
    wait_out(1)
    wsum(1)
    fire_out(1, STEPS - 1)
    wait_out(0)
    wait_out(1)


@jax.jit
def _sc_sample(loc, table):
    mesh = plsc.VectorSubcoreMesh(core_axis_name="c", subcore_axis_name="s",
                                  num_cores=NC, num_subcores=NS)
    f = pl.kernel(
        _tec_body,
        out_type=jax.ShapeDtypeStruct((N_PTS * ODIM,), jnp.float32),
        mesh=mesh,
        scratch_types=[
            pltpu.VMEM((2, B, 3), jnp.float32),
            pltpu.VMEM((2 * 3, B), jnp.int32),
            pltpu.VMEM((2 * 12, B), jnp.float32),
            pltpu.VMEM((2 * 3 * B, QW), jnp.int32),
            pltpu.VMEM((B * ODIM,), jnp.float32),
            pltpu.VMEM((B * ODIM,), jnp.float32),
            pltpu.SemaphoreType.DMA,
            pltpu.SemaphoreType.DMA,
            pltpu.SemaphoreType.DMA,
            pltpu.SemaphoreType.DMA,
            pltpu.SemaphoreType.DMA,
            pltpu.SemaphoreType.DMA,
        ],
        compiler_params=pltpu.CompilerParams(use_tc_tiling_on_sc=False,
                                             needs_layout_passes=False),
    )
    return f(loc, table)


BY = 8
NYB = RES // BY
OROWS = BY * RES * QW // 128


def _quad_body(a_ref, b_ref, o_ref):
    yblk = pl.program_id(1)
    a = a_ref[0]
    b = b_ref[0]
    last = yblk == (NYB - 1)
    brow = jnp.where(last, a[:, BY - 1:BY, :], b[:, 0:1, :])
    ay = jnp.concatenate([a[:, 1:, :], brow], axis=1)
    ax = jnp.concatenate([a[:, :, 1:], a[:, :, RES - 1:RES]], axis=2)
    axy = jnp.concatenate([ay[:, :, 1:], ay[:, :, RES - 1:RES]], axis=2)

    def pack(s):
        u = jax.lax.bitcast_convert_type(s, jnp.uint32)
        r = (u + jnp.uint32(0x7FFF) + ((u >> 16) & jnp.uint32(1))) >> 16
        r = r.reshape(DIM // 2, 2, BY, RES)
        return r[:, 0] | (r[:, 1] << 16)

    w = jnp.stack([pack(a), pack(ax), pack(ay), pack(axy)], axis=0)
    w = w.reshape(4, DIM // 2, BY, RES // 4, 4)
    w = jnp.transpose(w, (2, 3, 4, 0, 1)).reshape(OROWS, 128)
    o_ref[0] = jax.lax.bitcast_convert_type(w, jnp.int32)


@jax.jit
def _build_table(triplane):
    f = pl.pallas_call(
        _quad_body,
        grid=(3, NYB),
        in_specs=[
            pl.BlockSpec((1, DIM, BY, RES), lambda p, y: (p, 0, y, 0)),
            pl.BlockSpec((1, DIM, BY, RES),
                         lambda p, y: (p, 0, jnp.minimum(y + 1, NYB - 1), 0)),
        ],
        out_specs=pl.BlockSpec((1, OROWS, 128),
                               lambda p, y: (p * NYB + y, 0, 0)),
        out_shape=jax.ShapeDtypeStruct((3 * NYB, OROWS, 128), jnp.int32),
        compiler_params=pltpu.CompilerParams(
            dimension_semantics=("parallel", "arbitrary")),
    )
    return f(triplane, triplane).reshape(3 * RES * RES, QW)


def kernel(loc, triplane):
    tq = _build_table(triplane)
    out = _sc_sample(loc, tq)
    return out.reshape(N_PTS, ODIM)


# speedup vs baseline: 1.1244x; 1.0613x over previous
import jax
import jax.numpy as jnp
from jax import lax
from jax.experimental import pallas as pl
from jax.experimental.pallas import tpu as pltpu
from jax.experimental.pallas import tpu_sc as plsc

RES = 512
DIM = 16
ODIM = 3 * DIM
QW = 32
N_PTS = 2097152
NC, NS, LANES = 2, 16, 16
NW = NC * NS
PTS_W = N_PTS // NW
B = 128
STEPS = PTS_W // B
HALF = STEPS // 2
GROUPS = B // LANES


def _pix(c):
    p = jnp.clip(c * 128.0 + 255.5, 0.0, float(RES - 1))
    i0 = p.astype(jnp.int32)
    f = p - i0.astype(jnp.float32)
    return i0, f


def _tec_body(locT, table, out, cbuf, idxv, wv, gath, ob0, ob1,
              semc0, semc1, semg0, semg1, semo0, semo1):
    wid = lax.axis_index("s") * NC + lax.axis_index("c")
    semc = (semc0, semc1)
    semg = (semg0, semg1)
    semo = (semo0, semo1)
    obs = (ob0, ob1)

    def fire_coords(q, s):
        base = (wid * PTS_W + s * B) * 3
        pltpu.async_copy(locT.at[pl.ds(base, B * 3)], cbuf.at[q], semc[q])

    def wait_coords(q):
        pltpu.make_async_copy(locT.at[pl.ds(0, B * 3)], cbuf.at[q],
                              semc[q]).wait()

    def compute_idx(q):
        for g in range(GROUPS):
            sl = pl.ds(g * LANES, LANES)
            qv = jnp.full((LANES,), q, jnp.int32)
            p3 = (lax.iota(jnp.int32, LANES) + g * LANES) * 3
            x = plsc.load_gather(cbuf, (qv, p3))
            y = plsc.load_gather(cbuf, (qv, p3 + 1))
            z = plsc.load_gather(cbuf, (qv, p3 + 2))
            xi0, xf = _pix(x)
            yi0, yf = _pix(y)
            zi0, zf = _pix(z)
            specs = ((xi0, xf, yi0, yf),
                     (yi0, yf, zi0, zf),
                     (xi0, xf, zi0, zf))
            for p, (ci0, cf, ri0, rf) in enumerate(specs):
                idxv[q * 3 + p, sl] = (p * RES * RES) + ri0 * RES + ci0
                cw0 = 1.0 - cf
                rw0 = 1.0 - rf
                o = q * 12 + 4 * p
                wv[o + 0, sl] = rw0 * cw0
                wv[o + 1, sl] = rw0 * cf
                wv[o + 2, sl] = rf * cw0
                wv[o + 3, sl] = rf * cf

    def fire_gathers(q):
        for p in range(3):
            r = q * 3 + p
            pltpu.async_copy(table.at[idxv.at[r]],
                             gath.at[pl.ds(r * B, B)], semg[q])

    def wait_gathers(q):
        for p in range(3):
            r = q * 3 + p
            pltpu.make_async_copy(table.at[idxv.at[r]],
                                  gath.at[pl.ds(r * B, B)], semg[q]).wait()

    def wsum(q):
        ob = obs[q]

        def wsum_fn(g, _):
            sl = pl.ds(g * LANES, LANES)
            pid = lax.iota(jnp.int32, LANES) + g * LANES
            for pi in range(3):
                wp = [plsc.pack(wv[q * 12 + 4 * pi + t, sl],
                                wv[q * 12 + 4 * pi + t, sl],
                                format=plsc.PackFormat.INTERLEAVED)
                      for t in range(4)]
                row = (q * 3 + pi) * B + pid
                for cp in range(DIM // 2):
                    acc = None
                    for t in range(4):
                        cv = jnp.full((LANES,), t * (DIM // 2) + cp, jnp.int32)
                        v = plsc.bitcast(plsc.load_gather(gath, (row, cv)),
                                         jnp.bfloat16)
                        term = v * wp[t]
                        acc = term if acc is None else acc + term
                    a, b = plsc.unpack(acc,
                                       format=plsc.PackFormat.INTERLEAVED)
                    ov = pid * ODIM + (pi * DIM + 2 * cp)
                    plsc.store_scatter(ob, (ov,), a)
                    plsc.store_scatter(ob, (ov + 1,), b)
            return 0

        lax.fori_loop(0, GROUPS, wsum_fn, 0)

    def fire_out(q, s):
        base = (wid * PTS_W + s * B) * ODIM
        pltpu.async_copy(obs[q], out.at[pl.ds(base, B * ODIM)], semo[q])

    def wait_out(q):
        pltpu.make_async_copy(obs[q], out.at[pl.ds(0, B * ODIM)],
                              semo[q]).wait()

    fire_coords(0, 0)
    wait_coords(0)
    compute_idx(0)
    fire_gathers(0)
    fire_coords(1, 1)
    wait_coords(1)
    compute_idx(1)
    fire_gathers(1)
    fire_coords(0, 2)
    wait_gathers(0)
    wsum(0)
    fire_out(0, 0)

    def pair_body(i2, _):
        a = 2 * i2 + 1
        wait_coords(0)
        compute_idx(0)
        fire_gathers(0)
        fire_coords(1, a + 2)
        wait_gathers(1)

        @pl.when(i2 >= 1)
        def _():
            wait_out(1)

        wsum(1)
        fire_out(1, a)

        b = a + 1
        wait_coords(1)
        compute_idx(1)
        fire_gathers(1)

        @pl.when(i2 < HALF - 2)
        def _():
            fire_coords(0, b + 2)

        wait_gathers(0)
        wait_out(0)
        wsum(0)
        fire_out(0, b)
        return 0

    lax.fori_loop(0, HALF - 1, pair_body, 0)

    wait_gathers(1)
    wait_out(1)
    wsum(1)
    fire_out(1, STEPS - 1)
    wait_out(0)
    wait_out(1)


@jax.jit
def _sc_sample(locT, table):
    mesh = plsc.VectorSubcoreMesh(core_axis_name="c", subcore_axis_name="s",
                                  num_cores=NC, num_subcores=NS)
    f = pl.kernel(
        _tec_body,
        out_type=jax.ShapeDtypeStruct((N_PTS * ODIM,), jnp.float32),
        mesh=mesh,
        scratch_types=[
            pltpu.VMEM((2, 3 * B), jnp.float32),
            pltpu.VMEM((2 * 3, B), jnp.int32),
            pltpu.VMEM((2 * 12, B), jnp.float32),
            pltpu.VMEM((2 * 3 * B, QW), jnp.int32),
            pltpu.VMEM((B * ODIM,), jnp.float32),
            pltpu.VMEM((B * ODIM,), jnp.float32),
            pltpu.SemaphoreType.DMA,
            pltpu.SemaphoreType.DMA,
            pltpu.SemaphoreType.DMA,
            pltpu.SemaphoreType.DMA,
            pltpu.SemaphoreType.DMA,
            pltpu.SemaphoreType.DMA,
        ],
        compiler_params=pltpu.CompilerParams(use_tc_tiling_on_sc=False,
                                             needs_layout_passes=False),
    )
    return f(locT, table)


def kernel(loc, triplane):
    locT = loc.reshape(N_PTS * 3)
    t = jnp.transpose(triplane, (0, 2, 3, 1))
    t = t.astype(jnp.bfloat16)
    tx = jnp.concatenate([t[:, :, 1:], t[:, :, -1:]], axis=2)
    ty = jnp.concatenate([t[:, 1:], t[:, -1:]], axis=1)
    txy = jnp.concatenate([tx[:, 1:], tx[:, -1:]], axis=1)
    tq = jnp.concatenate([t, tx, ty, txy], axis=3)
    tq = jax.lax.bitcast_convert_type(
        tq.reshape(3 * RES * RES, QW, 2), jnp.int32)
    out = _sc_sample(locT, tq)
    return out.reshape(N_PTS, ODIM)


# speedup vs baseline: 1.1300x; 1.0050x over previous
import jax
import jax.numpy as jnp
from jax import lax
from jax.experimental import pallas as pl
from jax.experimental.pallas import tpu as pltpu
from jax.experimental.pallas import tpu_sc as plsc

RES = 512
DIM = 16
ODIM = 3 * DIM
QW = 32
N_PTS = 2097152
NC, NS, LANES = 2, 16, 16
NW = NC * NS
PTS_W = N_PTS // NW
B = 128
STEPS = PTS_W // B
HALF = STEPS // 2
GROUPS = B // LANES


def _pix(c):
    p = jnp.clip(c * 128.0 + 255.5, 0.0, float(RES - 1))
    i0 = p.astype(jnp.int32)
    f = p - i0.astype(jnp.float32)
    return i0, f


def _tec_body(locT, table, out, cbuf, idxv, wv, gath, ob0, ob1,
              semc0, semc1, semg0, semg1, semo0, semo1):
    wid = lax.axis_index("s") * NC + lax.axis_index("c")
    semc = (semc0, semc1)
    semg = (semg0, semg1)
    semo = (semo0, semo1)
    obs = (ob0, ob1)

    def fire_coords(q, s):
        base = (wid * PTS_W + s * B) * 3
        pltpu.async_copy(locT.at[pl.ds(base, B * 3)], cbuf.at[q], semc[q])

    def wait_coords(q):
        pltpu.make_async_copy(locT.at[pl.ds(0, B * 3)], cbuf.at[q],
                              semc[q]).wait()

    def compute_idx(q):
        for g in range(GROUPS):
            sl = pl.ds(g * LANES, LANES)
            qv = jnp.full((LANES,), q, jnp.int32)
            p3 = (lax.iota(jnp.int32, LANES) + g * LANES) * 3
            x = plsc.load_gather(cbuf, (qv, p3))
            y = plsc.load_gather(cbuf, (qv, p3 + 1))
            z = plsc.load_gather(cbuf, (qv, p3 + 2))
            xi0, xf = _pix(x)
            yi0, yf = _pix(y)
            zi0, zf = _pix(z)
            specs = ((xi0, xf, yi0, yf),
                     (yi0, yf, zi0, zf),
                     (xi0, xf, zi0, zf))
            for p, (ci0, cf, ri0, rf) in enumerate(specs):
                idxv[q * 3 + p, sl] = (p * RES * RES) + ri0 * RES + ci0
                cw0 = 1.0 - cf
                rw0 = 1.0 - rf
                o = q * 12 + 4 * p
                wv[o + 0, sl] = rw0 * cw0
                wv[o + 1, sl] = rw0 * cf
                wv[o + 2, sl] = rf * cw0
                wv[o + 3, sl] = rf * cf

    def fire_gathers(q):
        for p in range(3):
            r = q * 3 + p
            pltpu.async_copy(table.at[idxv.at[r]],
                             gath.at[pl.ds(r * B, B)], semg[q])

    def wait_gathers(q):
        for p in range(3):
            r = q * 3 + p
            pltpu.make_async_copy(table.at[idxv.at[r]],
                                  gath.at[pl.ds(r * B, B)], semg[q]).wait()

    def wsum(q):
        ob = obs[q]

        def wsum_fn(g, _):
            sl = pl.ds(g * LANES, LANES)
            pid = lax.iota(jnp.int32, LANES) + g * LANES
            for pi in range(3):
                w = [wv[q * 12 + 4 * pi + t, sl] for t in range(4)]
                row = (q * 3 + pi) * B + pid
                for cp in range(DIM // 2):
                    av = []
                    bv = []
                    for t in range(4):
                        cv = jnp.full((LANES,), t * (DIM // 2) + cp, jnp.int32)
                        v = plsc.load_gather(gath, (row, cv))
                        a, b = plsc.unpack(
                            plsc.bitcast(v, jnp.bfloat16),
                            format=plsc.PackFormat.INTERLEAVED)
                        av.append(a)
                        bv.append(b)
                    acca = av[0] * w[0] + av[1] * w[1] + av[2] * w[2] \
                        + av[3] * w[3]
                    accb = bv[0] * w[0] + bv[1] * w[1] + bv[2] * w[2] \
                        + bv[3] * w[3]
                    ov = pid * ODIM + (pi * DIM + 2 * cp)
                    plsc.store_scatter(ob, (ov,), acca)
                    plsc.store_scatter(ob, (ov + 1,), accb)
            return 0

        lax.fori_loop(0, GROUPS, wsum_fn, 0)

    def fire_out(q, s):
        base = (wid * PTS_W + s * B) * ODIM
        pltpu.async_copy(obs[q], out.at[pl.ds(base, B * ODIM)], semo[q])

    def wait_out(q):
        pltpu.make_async_copy(obs[q], out.at[pl.ds(0, B * ODIM)],
                              semo[q]).wait()

    fire_coords(0, 0)
    wait_coords(0)
    compute_idx(0)
    fire_gathers(0)
    fire_coords(1, 1)
    wait_coords(1)
    compute_idx(1)
    fire_gathers(1)
    fire_coords(0, 2)
    wait_gathers(0)
    wsum(0)
    fire_out(0, 0)

    def pair_body(i2, _):
        a = 2 * i2 + 1
        wait_coords(0)
        compute_idx(0)
        fire_gathers(0)
        fire_coords(1, a + 2)
        wait_gathers(1)

        @pl.when(i2 >= 1)
        def _():
            wait_out(1)

        wsum(1)
        fire_out(1, a)

        b = a + 1
        wait_coords(1)
        compute_idx(1)
        fire_gathers(1)

        @pl.when(i2 < HALF - 2)
        def _():
            fire_coords(0, b + 2)

        wait_gathers(0)
        wait_out(0)
        wsum(0)
        fire_out(0, b)
        return 0

    lax.fori_loop(0, HALF - 1, pair_body, 0)

    wait_gathers(1)
    wait_out(1)
    wsum(1)
    fire_out(1, STEPS - 1)
    wait_out(0)
    wait_out(1)


@jax.jit
def _sc_sample(locT, table):
    mesh = plsc.VectorSubcoreMesh(core_axis_name="c", subcore_axis_name="s",
                                  num_cores=NC, num_subcores=NS)
    f = pl.kernel(
        _tec_body,
        out_type=jax.ShapeDtypeStruct((N_PTS * ODIM,), jnp.float32),
        mesh=mesh,
        scratch_types=[
            pltpu.VMEM((2, 3 * B), jnp.float32),
            pltpu.VMEM((2 * 3, B), jnp.int32),
            pltpu.VMEM((2 * 12, B), jnp.float32),
            pltpu.VMEM((2 * 3 * B, QW), jnp.int32),
            pltpu.VMEM((B * ODIM,), jnp.float32),
            pltpu.VMEM((B * ODIM,), jnp.float32),
            pltpu.SemaphoreType.DMA,
            pltpu.SemaphoreType.DMA,
            pltpu.SemaphoreType.DMA,
            pltpu.SemaphoreType.DMA,
            pltpu.SemaphoreType.DMA,
            pltpu.SemaphoreType.DMA,
        ],
        compiler_params=pltpu.CompilerParams(use_tc_tiling_on_sc=False,
                                             needs_layout_passes=False),
    )
    return f(locT, table)


def kernel(loc, triplane):
    locT = loc.reshape(N_PTS * 3)
    t = jnp.transpose(triplane, (0, 2, 3, 1))
    t = t.astype(jnp.bfloat16)
    tx = jnp.concatenate([t[:, :, 1:], t[:, :, -1:]], axis=2)
    ty = jnp.concatenate([t[:, 1:], t[:, -1:]], axis=1)
    txy = jnp.concatenate([tx[:, 1:], tx[:, -1:]], axis=1)
    tq = jnp.concatenate([t, tx, ty, txy], axis=3)
    tq = jax.lax.bitcast_convert_type(
        tq.reshape(3 * RES * RES, QW, 2), jnp.int32)
    out = _sc_sample(locT, tq)
    return out.reshape(N_PTS, ODIM)


# speedup vs baseline: 1.4611x; 1.2929x over previous
import jax
import jax.numpy as jnp
from jax import lax
from jax.experimental import pallas as pl
from jax.experimental.pallas import tpu as pltpu
from jax.experimental.pallas import tpu_sc as plsc

RES = 512
DIM = 16
ODIM = 3 * DIM
QW = 32
N_PTS = 2097152
NC, NS, LANES = 2, 16, 16
NW = NC * NS
PTS_W = N_PTS // NW
B = 128
STEPS = PTS_W // B
HALF = STEPS // 2
GROUPS = B // LANES


def _pix(c):
    p = jnp.clip(c * 128.0 + 255.5, 0.0, float(RES - 1))
    i0 = p.astype(jnp.int32)
    f = p - i0.astype(jnp.float32)
    return i0, f


def _tec_body(locT, table, out, cbuf, idxv, wv, gath, ob0, ob1,
              semc0, semc1, semg0, semg1, semo0, semo1):
    wid = lax.axis_index("s") * NC + lax.axis_index("c")
    semc = (semc0, semc1)
    semg = (semg0, semg1)
    semo = (semo0, semo1)
    obs = (ob0, ob1)

    def fire_coords(q, s):
        base = wid * PTS_W + s * B
        pltpu.async_copy(locT.at[:, pl.ds(base, B)], cbuf.at[q], semc[q])

    def wait_coords(q):
        pltpu.make_async_copy(locT.at[:, pl.ds(0, B)], cbuf.at[q],
                              semc[q]).wait()

    def compute_idx(q):
        for g in range(GROUPS):
            sl = pl.ds(g * LANES, LANES)
            x = cbuf[q, 0, sl]
            y = cbuf[q, 1, sl]
            z = cbuf[q, 2, sl]
            xi0, xf = _pix(x)
            yi0, yf = _pix(y)
            zi0, zf = _pix(z)
            specs = ((xi0, xf, yi0, yf),
                     (yi0, yf, zi0, zf),
                     (xi0, xf, zi0, zf))
            for p, (ci0, cf, ri0, rf) in enumerate(specs):
                idxv[q * 3 + p, sl] = (p * RES * RES) + ri0 * RES + ci0
                cw0 = 1.0 - cf
                rw0 = 1.0 - rf
                o = q * 12 + 4 * p
                wv[o + 0, sl] = rw0 * cw0
                wv[o + 1, sl] = rw0 * cf
                wv[o + 2, sl] = rf * cw0
                wv[o + 3, sl] = rf * cf

    def fire_gathers(q):
        for p in range(3):
            r = q * 3 + p
            pltpu.async_copy(table.at[idxv.at[r]],
                             gath.at[pl.ds(r * B, B)], semg[q])

    def wait_gathers(q):
        for p in range(3):
            r = q * 3 + p
            pltpu.make_async_copy(table.at[idxv.at[r]],
                                  gath.at[pl.ds(r * B, B)], semg[q]).wait()

    def wsum(q):
        ob = obs[q]

        def wsum_fn(g, _):
            sl = pl.ds(g * LANES, LANES)
            pid = lax.iota(jnp.int32, LANES) + g * LANES
            for pi in range(3):
                wp = [plsc.pack(wv[q * 12 + 4 * pi + t, sl],
                                wv[q * 12 + 4 * pi + t, sl],
                                format=plsc.PackFormat.INTERLEAVED)
                      for t in range(4)]
                row = (q * 3 + pi) * B + pid
                for cp in range(DIM // 2):
                    acc = None
                    for t in range(4):
                        cv = jnp.full((LANES,), t * (DIM // 2) + cp, jnp.int32)
                        v = plsc.bitcast(plsc.load_gather(gath, (row, cv)),
                                         jnp.bfloat16)
                        term = v * wp[t]
                        acc = term if acc is None else acc + term
                    a, b = plsc.unpack(acc,
                                       format=plsc.PackFormat.INTERLEAVED)
                    ov = pid * ODIM + (pi * DIM + 2 * cp)
                    plsc.store_scatter(ob, (ov,), a)
                    plsc.store_scatter(ob, (ov + 1,), b)
            return 0

        lax.fori_loop(0, GROUPS, wsum_fn, 0)

    def fire_out(q, s):
        base = (wid * PTS_W + s * B) * ODIM
        pltpu.async_copy(obs[q], out.at[pl.ds(base, B * ODIM)], semo[q])

    def wait_out(q):
        pltpu.make_async_copy(obs[q], out.at[pl.ds(0, B * ODIM)],
                              semo[q]).wait()

    fire_coords(0, 0)
    wait_coords(0)
    compute_idx(0)
    fire_gathers(0)
    fire_coords(1, 1)
    wait_coords(1)
    compute_idx(1)
    fire_gathers(1)
    fire_coords(0, 2)
    wait_gathers(0)
    wsum(0)
    fire_out(0, 0)

    def pair_body(i2, _):
        a = 2 * i2 + 1
        wait_coords(0)
        compute_idx(0)
        fire_gathers(0)
        fire_coords(1, a + 2)
        wait_gathers(1)

        @pl.when(i2 >= 1)
        def _():
            wait_out(1)

        wsum(1)
        fire_out(1, a)

        b = a + 1
        wait_coords(1)
        compute_idx(1)
        fire_gathers(1)

        @pl.when(i2 < HALF - 2)
        def _():
            fire_coords(0, b + 2)

        wait_gathers(0)
        wait_out(0)
        wsum(0)
        fire_out(0, b)
        return 0

    lax.fori_loop(0, HALF - 1, pair_body, 0)

    wait_gathers(1)
    wait_out(1)
    wsum(1)
    fire_out(1, STEPS - 1)
    wait_out(0)
    wait_out(1)


@jax.jit
def _sc_sample(locT, table):
    mesh = plsc.VectorSubcoreMesh(core_axis_name="c", subcore_axis_name="s",
                                  num_cores=NC, num_subcores=NS)
    f = pl.kernel(
        _tec_body,
        out_type=jax.ShapeDtypeStruct((N_PTS * ODIM,), jnp.float32),
        mesh=mesh,
        scratch_types=[
            pltpu.VMEM((2, 3, B), jnp.float32),
            pltpu.VMEM((2 * 3, B), jnp.int32),
            pltpu.VMEM((2 * 12, B), jnp.float32),
            pltpu.VMEM((2 * 3 * B, QW), jnp.int32),
            pltpu.VMEM((B * ODIM,), jnp.float32),
            pltpu.VMEM((B * ODIM,), jnp.float32),
            pltpu.SemaphoreType.DMA,
            pltpu.SemaphoreType.DMA,
            pltpu.SemaphoreType.DMA,
            pltpu.SemaphoreType.DMA,
            pltpu.SemaphoreType.DMA,
            pltpu.SemaphoreType.DMA,
        ],
        compiler_params=pltpu.CompilerParams(use_tc_tiling_on_sc=False,
                                             needs_layout_passes=False),
    )
    return f(locT, table)


def kernel(loc, triplane):
    locT = jnp.transpose(loc)
    t = jnp.transpose(triplane, (0, 2, 3, 1))
    t = t.astype(jnp.bfloat16)
    tx = jnp.concatenate([t[:, :, 1:], t[:, :, -1:]], axis=2)
    ty = jnp.concatenate([t[:, 1:], t[:, -1:]], axis=1)
    txy = jnp.concatenate([tx[:, 1:], tx[:, -1:]], axis=1)
    tq = jnp.concatenate([t, tx, ty, txy], axis=3)
    tq = jax.lax.bitcast_convert_type(
        tq.reshape(3 * RES * RES, QW, 2), jnp.int32)
    out = _sc_sample(locT, tq)
    return out.reshape(N_PTS, ODIM)
